# baseline (speedup 1.0000x reference)
import functools

import jax
import jax.numpy as jnp
from jax import lax
from jax.experimental import pallas as pl
from jax.experimental.pallas import tpu as pltpu
from jax.experimental.pallas import tpu_sc as plsc

EMBED = 64
SEQ = 7
NC = 2
NS = 16
NW = NC * NS
CHUNK = 112
NBUF = 4


def _emb_body(idx_hbm, table_hbm, out_hbm,
              idx_v, b0, b1, b2, b3, g0, g1, g2, g3, o0, o1, o2, o3):
    wid = lax.axis_index("s") * NC + lax.axis_index("c")
    nchunk = idx_hbm.shape[1]
    rows_per_w = nchunk * CHUNK
    base = wid * rows_per_w
    bufs = [b0, b1, b2, b3]
    gsems = [g0, g1, g2, g3]
    osems = [o0, o1, o2, o3]

    pltpu.sync_copy(idx_hbm.at[wid], idx_v)

    def gather_cp(c, b):
        return pltpu.make_async_copy(table_hbm.at[idx_v.at[c]], bufs[b], gsems[b])

    def out_cp(c, b):
        dst = out_hbm.at[pl.ds(base + c * CHUNK, CHUNK)]
        return pltpu.make_async_copy(bufs[b], dst, osems[b])

    gather_cp(0, 0).start()
    gather_cp(1, 1).start()

    def loop_body(g, carry):
        for b in range(NBUF):
            c = NBUF * g + b
            gather_cp(c, b).wait()
            out_cp(c, b).start()
            b2_ = (b + 2) % NBUF

            @pl.when(c >= 2)
            def _():
                out_cp(c - 2, b2_).wait()

            @pl.when(c + 2 < NBUF * num_iters)
            def _():
                gather_cp(c + 2, b2_).start()
        return carry

    num_iters = nchunk // NBUF
    lax.fori_loop(0, num_iters, loop_body, 0)

    out_cp(nchunk - 2, (nchunk - 2) % NBUF).wait()
    out_cp(nchunk - 1, (nchunk - 1) % NBUF).wait()


@jax.jit
def kernel(x, token_table, pos_table):
    batch, seq = x.shape
    total = batch * seq
    nchunk = total // (NW * CHUNK)
    vocab = token_table.shape[0]
    xi = x.astype(jnp.int32)
    tblq = token_table.reshape(vocab // 2, 2 * EMBED)
    idx = (xi >> 1).reshape(NW, nchunk, CHUNK)

    mesh = plsc.VectorSubcoreMesh(core_axis_name="c", subcore_axis_name="s")
    emb = pl.kernel(
        _emb_body,
        mesh=mesh,
        compiler_params=pltpu.CompilerParams(use_tc_tiling_on_sc=False),
        out_type=jax.ShapeDtypeStruct((total, 2 * EMBED), jnp.float32),
        scratch_types=[
            pltpu.VMEM((nchunk, CHUNK), jnp.int32),
        ] + [pltpu.VMEM((CHUNK, 2 * EMBED), jnp.float32)] * NBUF
          + [pltpu.SemaphoreType.DMA] * (2 * NBUF),
    )
    raw = emb(idx, tblq)
    half = (xi & 1).reshape(total)
    sel = jnp.where(half[:, None] == 1, raw[:, EMBED:], raw[:, :EMBED])
    return sel.reshape(batch, seq, EMBED) + pos_table[None, :, :]


# speedup vs baseline: 1.2121x; 1.2121x over previous
import functools

import jax
import jax.numpy as jnp
from jax import lax
from jax.experimental import pallas as pl
from jax.experimental.pallas import tpu as pltpu
from jax.experimental.pallas import tpu_sc as plsc

EMBED = 64
SEQ = 7
NC = 2
NS = 16
NW = NC * NS
CHUNK = 128
NBUF = 4
LANES = 16


def _emb_body(idx_hbm, pos_hbm, table_hbm, out_hbm,
              idx_v, pos_v, b0, b1, b2, b3,
              g0, g1, g2, g3, o0, o1, o2, o3):
    wid = lax.axis_index("s") * NC + lax.axis_index("c")
    nchunk = idx_hbm.shape[1]
    batch = out_hbm.shape[1]
    base = wid * nchunk * CHUNK
    bufs = [b0, b1, b2, b3]
    gsems = [g0, g1, g2, g3]
    osems = [o0, o1, o2, o3]

    pltpu.sync_copy(idx_hbm.at[wid], idx_v)
    pltpu.sync_copy(pos_hbm, pos_v)

    def gather_cp(c, b):
        return pltpu.make_async_copy(table_hbm.at[idx_v.at[c]], bufs[b], gsems[b])

    def out_cp(c, b):
        f0 = base + c * CHUNK
        dst = out_hbm.at[f0 // batch, pl.ds(f0 % batch, CHUNK)]
        return pltpu.make_async_copy(bufs[b], dst, osems[b])

    def add_pos(c, b):
        buf = bufs[b]
        s = (base + c * CHUNK) // batch
        pos_regs = [pos_v[s, pl.ds(v * LANES, LANES)] for v in range(EMBED // LANES)]

        def add_body(r, inner):
            for v in range(EMBED // LANES):
                sl = pl.ds(v * LANES, LANES)
                buf[r, sl] = buf[r, sl] + pos_regs[v]
            return inner

        lax.fori_loop(0, CHUNK, add_body, 0)

    gather_cp(0, 0).start()
    gather_cp(1, 1).start()

    def loop_body(g, carry):
        for b in range(NBUF):
            c = NBUF * g + b
            gather_cp(c, b).wait()
            add_pos(c, b)
            out_cp(c, b).start()
            b2_ = (b + 2) % NBUF

            @pl.when(c >= 2)
            def _():
                out_cp(c - 2, b2_).wait()

            @pl.when(c + 2 < NBUF * num_iters)
            def _():
                gather_cp(c + 2, b2_).start()
        return carry

    num_iters = nchunk // NBUF
    lax.fori_loop(0, num_iters, loop_body, 0)

    out_cp(nchunk - 2, (nchunk - 2) % NBUF).wait()
    out_cp(nchunk - 1, (nchunk - 1) % NBUF).wait()


@jax.jit
def kernel(x, token_table, pos_table):
    batch, seq = x.shape
    total = batch * seq
    nchunk = total // (NW * CHUNK)
    idx = x.astype(jnp.int32).T.reshape(NW, nchunk, CHUNK)

    mesh = plsc.VectorSubcoreMesh(core_axis_name="c", subcore_axis_name="s")
    emb = pl.kernel(
        _emb_body,
        mesh=mesh,
        compiler_params=pltpu.CompilerParams(use_tc_tiling_on_sc=False),
        out_type=jax.ShapeDtypeStruct((seq, batch, EMBED), jnp.float32),
        scratch_types=[
            pltpu.VMEM((nchunk, CHUNK), jnp.int32),
            pltpu.VMEM((SEQ, EMBED), jnp.float32),
        ] + [pltpu.VMEM((CHUNK, EMBED), jnp.float32)] * NBUF
          + [pltpu.SemaphoreType.DMA] * (2 * NBUF),
    )
    out = emb(idx, pos_table, token_table)
    return out.transpose(1, 0, 2)


# speedup vs baseline: 1.2159x; 1.0032x over previous
import jax
import jax.numpy as jnp
from jax import lax
from jax.experimental import pallas as pl
from jax.experimental.pallas import tpu as pltpu
from jax.experimental.pallas import tpu_sc as plsc

EMBED = 64
SEQ = 7
NC = 2
NS = 16
NW = NC * NS
CHUNK = 128
NBUF = 4
LANES = 16


def _emb_body(idx_hbm, pos_hbm, table_hbm, out_hbm,
              idx_v, pos_v, b0, b1, b2, b3,
              g0, g1, g2, g3, o0, o1, o2, o3):
    wid = lax.axis_index("s") * NC + lax.axis_index("c")
    nchunk = idx_hbm.shape[1]
    batch = out_hbm.shape[1]
    base = wid * nchunk * CHUNK
    bufs = [b0, b1, b2, b3]
    gsems = [g0, g1, g2, g3]
    osems = [o0, o1, o2, o3]

    pltpu.sync_copy(idx_hbm.at[wid], idx_v)
    pltpu.sync_copy(pos_hbm, pos_v)

    def gather_cp(c, b):
        return pltpu.make_async_copy(table_hbm.at[idx_v.at[c]], bufs[b], gsems[b])

    def out_cp(c, b):
        f0 = base + c * CHUNK
        dst = out_hbm.at[f0 // batch, pl.ds(f0 % batch, CHUNK)]
        return pltpu.make_async_copy(bufs[b], dst, osems[b])

    def add_pos(c, b):
        buf = bufs[b]
        s = (base + c * CHUNK) // batch
        pos_regs = [pos_v[s, pl.ds(v * LANES, LANES)] for v in range(EMBED // LANES)]

        def add_body(r, inner):
            for v in range(EMBED // LANES):
                sl = pl.ds(v * LANES, LANES)
                buf[r, sl] = buf[r, sl] + pos_regs[v]
            return inner

        lax.fori_loop(0, CHUNK, add_body, 0)

    gather_cp(0, 0).start()
    gather_cp(1, 1).start()

    def loop_body(g, carry):
        for b in range(NBUF):
            c = NBUF * g + b
            gather_cp(c, b).wait()
            add_pos(c, b)
            out_cp(c, b).start()
            b2_ = (b + 2) % NBUF

            @pl.when(c >= 2)
            def _():
                out_cp(c - 2, b2_).wait()

            @pl.when(c + 2 < NBUF * num_iters)
            def _():
                gather_cp(c + 2, b2_).start()
        return carry

    num_iters = nchunk // NBUF
    lax.fori_loop(0, num_iters, loop_body, 0)

    out_cp(nchunk - 2, (nchunk - 2) % NBUF).wait()
    out_cp(nchunk - 1, (nchunk - 1) % NBUF).wait()


@jax.jit
def kernel(x, token_table, pos_table):
    batch, seq = x.shape
    total = batch * seq
    nchunk = total // (NW * CHUNK)
    idx = x.astype(jnp.int32).T.reshape(NW, nchunk, CHUNK)

    mesh = plsc.VectorSubcoreMesh(core_axis_name="c", subcore_axis_name="s")
    emb = pl.kernel(
        _emb_body,
        mesh=mesh,
        compiler_params=pltpu.CompilerParams(use_tc_tiling_on_sc=False),
        out_type=jax.ShapeDtypeStruct((seq, batch, EMBED), jnp.float32),
        scratch_types=[
            pltpu.VMEM((nchunk, CHUNK), jnp.int32),
            pltpu.VMEM((SEQ, EMBED), jnp.float32),
        ] + [pltpu.VMEM((CHUNK, EMBED), jnp.float32)] * NBUF
          + [pltpu.SemaphoreType.DMA] * (2 * NBUF),
    )
    out = emb(idx, pos_table, token_table)
    return out.transpose(1, 0, 2)
